# baseline (speedup 1.0000x reference)
import jax
import jax.numpy as jnp
from jax import lax
from jax.experimental import pallas as pl
from jax.experimental.pallas import tpu as pltpu
from jax.experimental.pallas import tpu_sc as plsc

BATCH = 128
N = 32768
KSEL = 3277
NC = 2
NS = 16
NW = NC * NS
ROWS_PER_W = BATCH // NW
L = 16
NV = N // L
NBINS = 256
HIST_STRIDE = NBINS + 1
HIST_WORDS = L * HIST_STRIDE
CH = 2048
NCH = N // CH
RING = 4


def _body(x_hbm, duty_hbm, out_hbm, x_v, keys_v, bf_v, hist_v, total_v,
          ring_v, in_sem, out_sem):
    wid = lax.axis_index("s") * NC + lax.axis_index("c")
    lanes = lax.iota(jnp.int32, L)
    lane_off = lanes * HIST_STRIDE
    ones = jnp.ones((L,), jnp.int32)
    zeros_i = jnp.zeros((L,), jnp.int32)

    pltpu.sync_copy(duty_hbm, x_v)
    td = jnp.float32(KSEL / N)

    @plsc.parallel_loop(0, N, L, unroll=8)
    def bf_step(i):
        bf_v[pl.ds(i, L)] = jnp.exp(td - x_v[pl.ds(i, L)])

    def zero_hist():
        @plsc.parallel_loop(0, HIST_WORDS, L, unroll=8)
        def z_step(j):
            hist_v[pl.ds(j, L)] = zeros_i

    def reduce_and_clear_hist():
        @plsc.parallel_loop(0, NBINS, L, unroll=4)
        def t_zero(j):
            total_v[pl.ds(j, L)] = zeros_i
        for l in range(L):
            @plsc.parallel_loop(0, NBINS, L, unroll=4)
            def t_acc(j, l=l):
                sl = pl.ds(l * HIST_STRIDE + j, L)
                plsc.addupdate(total_v.at[pl.ds(j, L)], hist_v[sl])
                hist_v[sl] = zeros_i

    def bin_search(r):
        reduce_and_clear_hist()
        C = jnp.int32(0)
        bsum = jnp.int32(0)
        ssum = jnp.int32(0)
        for g in range(15, -1, -1):
            base = g * L
            v = total_v[pl.ds(base, L)]
            rev = lax.rev(v, (0,))
            cs = plsc.cumsum(rev)
            up = C + cs
            s_strict = up - rev
            m = jnp.logical_and(s_strict < r, up >= r)
            binvec = (base + (L - 1)) - lanes
            bsum = bsum + jnp.sum(jnp.where(m, binvec, 0))
            ssum = ssum + jnp.sum(jnp.where(m, s_strict, 0))
            C = C + cs[15]
        return bsum, r - ssum

    zero_hist()
    pltpu.sync_copy(x_hbm.at[wid * ROWS_PER_W], x_v)

    def row_step(ri, c):
        row = wid * ROWS_PER_W + ri

        @plsc.parallel_loop(0, N, L, unroll=8)
        def p1_step(i):
            sl = pl.ds(i, L)
            b = x_v[sl] * bf_v[sl]
            bi = lax.bitcast_convert_type(b, jnp.int32)
            t = lax.shift_right_arithmetic(bi, 31)
            key = lax.bitwise_xor(bi, lax.bitwise_and(t, jnp.int32(0x7FFFFFFF)))
            keys_v[sl] = key
            bucket = lax.shift_right_arithmetic(key, 24) + 128
            plsc.addupdate_scatter(hist_v, [lane_off + bucket], ones)

        b1, r = bin_search(jnp.int32(KSEL))
        prefix = b1 - 128

        def radix_pass(shift_hi, shift_lo, prefix, r):
            @plsc.parallel_loop(0, N, L, unroll=8)
            def p_step(i):
                key = keys_v[pl.ds(i, L)]
                active = lax.shift_right_arithmetic(key, shift_hi) == prefix
                bucket = lax.bitwise_and(
                    lax.shift_right_arithmetic(key, shift_lo), jnp.int32(255))
                plsc.addupdate_scatter(
                    hist_v, [lane_off + bucket], ones, mask=active)

            return bin_search(r)

        b2, r = radix_pass(24, 16, prefix, r)
        prefix = prefix * 256 + b2
        b3, r = radix_pass(16, 8, prefix, r)
        prefix = prefix * 256 + b3
        b4, r = radix_pass(8, 0, prefix, r)
        kth = prefix * 256 + b4

        nxt = row + jnp.where(ri < ROWS_PER_W - 1, 1, 0)
        in_handles, out_handles = [], []
        for ci in range(NCH):
            base = ci * CH
            slot = ci % RING
            if ci >= RING:
                out_handles[ci - RING].wait()

            @plsc.parallel_loop(base, base + CH, L, unroll=8)
            def chunk_step(i, base=base, slot=slot):
                sl = pl.ds(i, L)
                m = keys_v[sl] >= kth
                ring_v[pl.ds(slot * CH + (i - base), L)] = jnp.where(
                    m, x_v[sl], jnp.float32(0.0))

            out_handles.append(pltpu.async_copy(
                ring_v.at[pl.ds(slot * CH, CH)],
                out_hbm.at[row, pl.ds(base, CH)], out_sem))
            if ci >= 1:
                pb = (ci - 1) * CH
                in_handles.append(pltpu.async_copy(
                    x_hbm.at[nxt, pl.ds(pb, CH)],
                    x_v.at[pl.ds(pb, CH)], in_sem))
        in_handles.append(pltpu.async_copy(
            x_hbm.at[nxt, pl.ds((NCH - 1) * CH, CH)],
            x_v.at[pl.ds((NCH - 1) * CH, CH)], in_sem))
        for h in out_handles[NCH - RING:]:
            h.wait()
        for h in in_handles:
            h.wait()
        return c

    lax.fori_loop(0, ROWS_PER_W, row_step, 0)


def kernel(x, duty_cycles):
    mesh = plsc.VectorSubcoreMesh(
        core_axis_name="c", subcore_axis_name="s",
        num_cores=NC, num_subcores=NS)
    f = pl.kernel(
        _body,
        out_type=jax.ShapeDtypeStruct((BATCH, N), jnp.float32),
        mesh=mesh,
        compiler_params=pltpu.CompilerParams(needs_layout_passes=False),
        scratch_types=[
            pltpu.VMEM((N,), jnp.float32),
            pltpu.VMEM((N,), jnp.int32),
            pltpu.VMEM((N,), jnp.float32),
            pltpu.VMEM((HIST_WORDS,), jnp.int32),
            pltpu.VMEM((NBINS,), jnp.int32),
            pltpu.VMEM((RING * CH,), jnp.float32),
            pltpu.SemaphoreType.DMA,
            pltpu.SemaphoreType.DMA,
        ],
    )
    return f(x, duty_cycles)


# speedup vs baseline: 1.1399x; 1.1399x over previous
import jax
import jax.numpy as jnp
from jax import lax
from jax.experimental import pallas as pl
from jax.experimental.pallas import tpu as pltpu
from jax.experimental.pallas import tpu_sc as plsc

BATCH = 128
N = 32768
KSEL = 3277
NC = 2
NS = 16
NW = NC * NS
ROWS_PER_W = BATCH // NW
L = 16
NV = N // L
NBINS = 256
HIST_STRIDE = NBINS + 1
HIST_WORDS = L * HIST_STRIDE
CH = 2048
NCH = N // CH
RING = 4


def _body(x_hbm, duty_hbm, out_hbm, x_v, keys_v, bf_v, hist_v, ring_v,
          in_sem, out_sem):
    wid = lax.axis_index("s") * NC + lax.axis_index("c")
    lanes = lax.iota(jnp.int32, L)
    lane_off = lanes * HIST_STRIDE
    ones = jnp.ones((L,), jnp.int32)
    zeros_i = jnp.zeros((L,), jnp.int32)

    pltpu.sync_copy(duty_hbm, x_v)
    td = jnp.float32(KSEL / N)

    @plsc.parallel_loop(0, N, L, unroll=8)
    def bf_step(i):
        bf_v[pl.ds(i, L)] = jnp.exp(td - x_v[pl.ds(i, L)])

    def zero_hist():
        @plsc.parallel_loop(0, HIST_WORDS, L, unroll=8)
        def z_step(j):
            hist_v[pl.ds(j, L)] = zeros_i

    def bin_search(r):
        def g_step(gi, carry):
            C, bsum, ssum = carry
            base = (15 - gi) * L
            v = hist_v[pl.ds(base, L)]
            hist_v[pl.ds(base, L)] = zeros_i
            for l in range(1, L):
                sl = pl.ds(l * HIST_STRIDE + base, L)
                v = v + hist_v[sl]
                hist_v[sl] = zeros_i
            rev = lax.rev(v, (0,))
            cs = plsc.cumsum(rev)
            up = C + cs
            s_strict = up - rev
            m = jnp.logical_and(s_strict < r, up >= r)
            binvec = (base + (L - 1)) - lanes
            bsum = bsum + jnp.sum(jnp.where(m, binvec, 0))
            ssum = ssum + jnp.sum(jnp.where(m, s_strict, 0))
            C = C + cs[15]
            return (C, bsum, ssum)

        C, bsum, ssum = lax.fori_loop(
            0, 16, g_step, (jnp.int32(0), jnp.int32(0), jnp.int32(0)))
        return bsum, r - ssum

    zero_hist()
    pltpu.sync_copy(x_hbm.at[wid * ROWS_PER_W], x_v)

    def row_step(ri, c):
        row = wid * ROWS_PER_W + ri

        @plsc.parallel_loop(0, N, L, unroll=8)
        def p1_step(i):
            sl = pl.ds(i, L)
            b = x_v[sl] * bf_v[sl]
            bi = lax.bitcast_convert_type(b, jnp.int32)
            t = lax.shift_right_arithmetic(bi, 31)
            key = lax.bitwise_xor(bi, lax.bitwise_and(t, jnp.int32(0x7FFFFFFF)))
            keys_v[sl] = key
            bucket = lax.shift_right_arithmetic(key, 24) + 128
            plsc.addupdate_scatter(hist_v, [lane_off + bucket], ones)

        b1, r = bin_search(jnp.int32(KSEL))
        prefix = b1 - 128

        def radix_pass(shift_hi, shift_lo, prefix, r):
            @plsc.parallel_loop(0, N, L, unroll=8)
            def p_step(i):
                key = keys_v[pl.ds(i, L)]
                active = lax.shift_right_arithmetic(key, shift_hi) == prefix
                bucket = lax.bitwise_and(
                    lax.shift_right_arithmetic(key, shift_lo), jnp.int32(255))
                plsc.addupdate_scatter(
                    hist_v, [lane_off + bucket], ones, mask=active)

            return bin_search(r)

        b2, r = radix_pass(24, 16, prefix, r)
        prefix = prefix * 256 + b2
        b3, r = radix_pass(16, 8, prefix, r)
        prefix = prefix * 256 + b3
        b4, r = radix_pass(8, 0, prefix, r)
        kth = prefix * 256 + b4

        nxt = row + jnp.where(ri < ROWS_PER_W - 1, 1, 0)
        in_handles, out_handles = [], []
        for ci in range(NCH):
            base = ci * CH
            slot = ci % RING
            if ci >= RING:
                out_handles[ci - RING].wait()

            @plsc.parallel_loop(base, base + CH, L, unroll=8)
            def chunk_step(i, base=base, slot=slot):
                sl = pl.ds(i, L)
                m = keys_v[sl] >= kth
                ring_v[pl.ds(slot * CH + (i - base), L)] = jnp.where(
                    m, x_v[sl], jnp.float32(0.0))

            out_handles.append(pltpu.async_copy(
                ring_v.at[pl.ds(slot * CH, CH)],
                out_hbm.at[row, pl.ds(base, CH)], out_sem))
            if ci >= 1:
                pb = (ci - 1) * CH
                in_handles.append(pltpu.async_copy(
                    x_hbm.at[nxt, pl.ds(pb, CH)],
                    x_v.at[pl.ds(pb, CH)], in_sem))
        in_handles.append(pltpu.async_copy(
            x_hbm.at[nxt, pl.ds((NCH - 1) * CH, CH)],
            x_v.at[pl.ds((NCH - 1) * CH, CH)], in_sem))
        for h in out_handles[NCH - RING:]:
            h.wait()
        for h in in_handles:
            h.wait()
        return c

    lax.fori_loop(0, ROWS_PER_W, row_step, 0)


def kernel(x, duty_cycles):
    mesh = plsc.VectorSubcoreMesh(
        core_axis_name="c", subcore_axis_name="s",
        num_cores=NC, num_subcores=NS)
    f = pl.kernel(
        _body,
        out_type=jax.ShapeDtypeStruct((BATCH, N), jnp.float32),
        mesh=mesh,
        compiler_params=pltpu.CompilerParams(needs_layout_passes=False),
        scratch_types=[
            pltpu.VMEM((N,), jnp.float32),
            pltpu.VMEM((N,), jnp.int32),
            pltpu.VMEM((N,), jnp.float32),
            pltpu.VMEM((HIST_WORDS,), jnp.int32),
            pltpu.VMEM((RING * CH,), jnp.float32),
            pltpu.SemaphoreType.DMA,
            pltpu.SemaphoreType.DMA,
        ],
    )
    return f(x, duty_cycles)


# speedup vs baseline: 1.1483x; 1.0074x over previous
import jax
import jax.numpy as jnp
from jax import lax
from jax.experimental import pallas as pl
from jax.experimental.pallas import tpu as pltpu
from jax.experimental.pallas import tpu_sc as plsc

BATCH = 128
N = 32768
KSEL = 3277
NC = 2
NS = 16
NW = NC * NS
ROWS_PER_W = BATCH // NW
L = 16
NV = N // L
NBINS = 256
HIST_STRIDE = NBINS + 1
HIST_WORDS = L * HIST_STRIDE
CH = 2048
NCH = N // CH
RING = 4


def _body(x_hbm, duty_hbm, out_hbm, x_v, keys_v, bf_v, hist_v, ring_v,
          in_sem, out_sem):
    wid = lax.axis_index("s") * NC + lax.axis_index("c")
    lanes = lax.iota(jnp.int32, L)
    lane_off = lanes * HIST_STRIDE
    ones = jnp.ones((L,), jnp.int32)
    zeros_i = jnp.zeros((L,), jnp.int32)

    pltpu.sync_copy(duty_hbm, x_v)
    td = jnp.float32(KSEL / N)

    @plsc.parallel_loop(0, N, L, unroll=16)
    def bf_step(i):
        bf_v[pl.ds(i, L)] = jnp.exp(td - x_v[pl.ds(i, L)])

    def zero_hist():
        @plsc.parallel_loop(0, HIST_WORDS, L, unroll=8)
        def z_step(j):
            hist_v[pl.ds(j, L)] = zeros_i

    def bin_search(r):
        def g_step(gi, carry):
            C, bsum, ssum = carry
            base = (15 - gi) * L
            v = hist_v[pl.ds(base, L)]
            hist_v[pl.ds(base, L)] = zeros_i
            for l in range(1, L):
                sl = pl.ds(l * HIST_STRIDE + base, L)
                v = v + hist_v[sl]
                hist_v[sl] = zeros_i
            rev = lax.rev(v, (0,))
            cs = plsc.cumsum(rev)
            up = C + cs
            s_strict = up - rev
            m = jnp.logical_and(s_strict < r, up >= r)
            binvec = (base + (L - 1)) - lanes
            bsum = bsum + jnp.sum(jnp.where(m, binvec, 0))
            ssum = ssum + jnp.sum(jnp.where(m, s_strict, 0))
            C = C + cs[15]
            return (C, bsum, ssum)

        C, bsum, ssum = lax.fori_loop(
            0, 16, g_step, (jnp.int32(0), jnp.int32(0), jnp.int32(0)))
        return bsum, r - ssum

    zero_hist()
    pltpu.sync_copy(x_hbm.at[wid * ROWS_PER_W], x_v)

    def row_step(ri, c):
        row = wid * ROWS_PER_W + ri

        @plsc.parallel_loop(0, N, L, unroll=16)
        def p1_step(i):
            sl = pl.ds(i, L)
            b = x_v[sl] * bf_v[sl]
            bi = lax.bitcast_convert_type(b, jnp.int32)
            t = lax.shift_right_arithmetic(bi, 31)
            key = lax.bitwise_xor(bi, lax.bitwise_and(t, jnp.int32(0x7FFFFFFF)))
            keys_v[sl] = key
            bucket = lax.shift_right_arithmetic(key, 24) + 128
            plsc.addupdate_scatter(hist_v, [lane_off + bucket], ones)

        b1, r = bin_search(jnp.int32(KSEL))
        prefix = b1 - 128

        def radix_pass(shift_hi, shift_lo, prefix, r):
            @plsc.parallel_loop(0, N, L, unroll=16)
            def p_step(i):
                key = keys_v[pl.ds(i, L)]
                active = lax.shift_right_arithmetic(key, shift_hi) == prefix
                bucket = lax.bitwise_and(
                    lax.shift_right_arithmetic(key, shift_lo), jnp.int32(255))
                plsc.addupdate_scatter(
                    hist_v, [lane_off + bucket], ones, mask=active)

            return bin_search(r)

        b2, r = radix_pass(24, 16, prefix, r)
        prefix = prefix * 256 + b2
        b3, r = radix_pass(16, 8, prefix, r)
        prefix = prefix * 256 + b3
        b4, r = radix_pass(8, 0, prefix, r)
        kth = prefix * 256 + b4

        nxt = row + jnp.where(ri < ROWS_PER_W - 1, 1, 0)
        in_handles, out_handles = [], []
        for ci in range(NCH):
            base = ci * CH
            slot = ci % RING
            if ci >= RING:
                out_handles[ci - RING].wait()

            @plsc.parallel_loop(base, base + CH, L, unroll=8)
            def chunk_step(i, base=base, slot=slot):
                sl = pl.ds(i, L)
                m = keys_v[sl] >= kth
                ring_v[pl.ds(slot * CH + (i - base), L)] = jnp.where(
                    m, x_v[sl], jnp.float32(0.0))

            out_handles.append(pltpu.async_copy(
                ring_v.at[pl.ds(slot * CH, CH)],
                out_hbm.at[row, pl.ds(base, CH)], out_sem))
            if ci >= 1:
                pb = (ci - 1) * CH
                in_handles.append(pltpu.async_copy(
                    x_hbm.at[nxt, pl.ds(pb, CH)],
                    x_v.at[pl.ds(pb, CH)], in_sem))
        in_handles.append(pltpu.async_copy(
            x_hbm.at[nxt, pl.ds((NCH - 1) * CH, CH)],
            x_v.at[pl.ds((NCH - 1) * CH, CH)], in_sem))
        for h in out_handles[NCH - RING:]:
            h.wait()
        for h in in_handles:
            h.wait()
        return c

    lax.fori_loop(0, ROWS_PER_W, row_step, 0)


def kernel(x, duty_cycles):
    mesh = plsc.VectorSubcoreMesh(
        core_axis_name="c", subcore_axis_name="s",
        num_cores=NC, num_subcores=NS)
    f = pl.kernel(
        _body,
        out_type=jax.ShapeDtypeStruct((BATCH, N), jnp.float32),
        mesh=mesh,
        compiler_params=pltpu.CompilerParams(needs_layout_passes=False),
        scratch_types=[
            pltpu.VMEM((N,), jnp.float32),
            pltpu.VMEM((N,), jnp.int32),
            pltpu.VMEM((N,), jnp.float32),
            pltpu.VMEM((HIST_WORDS,), jnp.int32),
            pltpu.VMEM((RING * CH,), jnp.float32),
            pltpu.SemaphoreType.DMA,
            pltpu.SemaphoreType.DMA,
        ],
    )
    return f(x, duty_cycles)


# speedup vs baseline: 1.1485x; 1.0002x over previous
import jax
import jax.numpy as jnp
from jax import lax
from jax.experimental import pallas as pl
from jax.experimental.pallas import tpu as pltpu
from jax.experimental.pallas import tpu_sc as plsc

BATCH = 128
N = 32768
KSEL = 3277
NC = 2
NS = 16
NW = NC * NS
ROWS_PER_W = BATCH // NW
L = 16
NV = N // L
NBINS = 256
HIST_STRIDE = NBINS + 1
HIST_WORDS = L * HIST_STRIDE
CH = 4096
NCH = N // CH
RING = 2


def _body(x_hbm, duty_hbm, out_hbm, x_v, keys_v, bf_v, hist_v, ring_v,
          in_sem, out_sem):
    wid = lax.axis_index("s") * NC + lax.axis_index("c")
    lanes = lax.iota(jnp.int32, L)
    lane_off = lanes * HIST_STRIDE
    ones = jnp.ones((L,), jnp.int32)
    zeros_i = jnp.zeros((L,), jnp.int32)

    pltpu.sync_copy(duty_hbm, x_v)
    td = jnp.float32(KSEL / N)

    @plsc.parallel_loop(0, N, L, unroll=16)
    def bf_step(i):
        bf_v[pl.ds(i, L)] = jnp.exp(td - x_v[pl.ds(i, L)])

    def zero_hist():
        @plsc.parallel_loop(0, HIST_WORDS, L, unroll=8)
        def z_step(j):
            hist_v[pl.ds(j, L)] = zeros_i

    def bin_search(r):
        def g_step(gi, carry):
            C, bsum, ssum = carry
            base = (15 - gi) * L
            v = hist_v[pl.ds(base, L)]
            hist_v[pl.ds(base, L)] = zeros_i
            for l in range(1, L):
                sl = pl.ds(l * HIST_STRIDE + base, L)
                v = v + hist_v[sl]
                hist_v[sl] = zeros_i
            rev = lax.rev(v, (0,))
            cs = plsc.cumsum(rev)
            up = C + cs
            s_strict = up - rev
            m = jnp.logical_and(s_strict < r, up >= r)
            binvec = (base + (L - 1)) - lanes
            bsum = bsum + jnp.sum(jnp.where(m, binvec, 0))
            ssum = ssum + jnp.sum(jnp.where(m, s_strict, 0))
            C = C + cs[15]
            return (C, bsum, ssum)

        C, bsum, ssum = lax.fori_loop(
            0, 16, g_step, (jnp.int32(0), jnp.int32(0), jnp.int32(0)))
        return bsum, r - ssum

    zero_hist()
    pltpu.sync_copy(x_hbm.at[wid * ROWS_PER_W], x_v)

    def row_step(ri, c):
        row = wid * ROWS_PER_W + ri

        @plsc.parallel_loop(0, N, L, unroll=16)
        def p1_step(i):
            sl = pl.ds(i, L)
            b = x_v[sl] * bf_v[sl]
            bi = lax.bitcast_convert_type(b, jnp.int32)
            t = lax.shift_right_arithmetic(bi, 31)
            key = lax.bitwise_xor(bi, lax.bitwise_and(t, jnp.int32(0x7FFFFFFF)))
            keys_v[sl] = key
            bucket = lax.shift_right_arithmetic(key, 24) + 128
            plsc.addupdate_scatter(hist_v, [lane_off + bucket], ones)

        b1, r = bin_search(jnp.int32(KSEL))
        prefix = b1 - 128

        def radix_pass(shift_hi, shift_lo, prefix, r):
            @plsc.parallel_loop(0, N, L, unroll=16)
            def p_step(i):
                key = keys_v[pl.ds(i, L)]
                active = lax.shift_right_arithmetic(key, shift_hi) == prefix
                bucket = lax.bitwise_and(
                    lax.shift_right_arithmetic(key, shift_lo), jnp.int32(255))
                plsc.addupdate_scatter(
                    hist_v, [lane_off + bucket], ones, mask=active)

            return bin_search(r)

        b2, r = radix_pass(24, 16, prefix, r)
        prefix = prefix * 256 + b2
        b3, r = radix_pass(16, 8, prefix, r)
        prefix = prefix * 256 + b3
        b4, r = radix_pass(8, 0, prefix, r)
        kth = prefix * 256 + b4

        nxt = row + jnp.where(ri < ROWS_PER_W - 1, 1, 0)
        in_handles, out_handles = [], []
        for ci in range(NCH):
            base = ci * CH
            slot = ci % RING
            if ci >= RING:
                out_handles[ci - RING].wait()

            @plsc.parallel_loop(base, base + CH, L, unroll=16)
            def chunk_step(i, base=base, slot=slot):
                sl = pl.ds(i, L)
                m = keys_v[sl] >= kth
                ring_v[pl.ds(slot * CH + (i - base), L)] = jnp.where(
                    m, x_v[sl], jnp.float32(0.0))

            out_handles.append(pltpu.async_copy(
                ring_v.at[pl.ds(slot * CH, CH)],
                out_hbm.at[row, pl.ds(base, CH)], out_sem))
            if ci >= 1:
                pb = (ci - 1) * CH
                in_handles.append(pltpu.async_copy(
                    x_hbm.at[nxt, pl.ds(pb, CH)],
                    x_v.at[pl.ds(pb, CH)], in_sem))
        in_handles.append(pltpu.async_copy(
            x_hbm.at[nxt, pl.ds((NCH - 1) * CH, CH)],
            x_v.at[pl.ds((NCH - 1) * CH, CH)], in_sem))
        for h in out_handles[NCH - RING:]:
            h.wait()
        for h in in_handles:
            h.wait()
        return c

    lax.fori_loop(0, ROWS_PER_W, row_step, 0)


def kernel(x, duty_cycles):
    mesh = plsc.VectorSubcoreMesh(
        core_axis_name="c", subcore_axis_name="s",
        num_cores=NC, num_subcores=NS)
    f = pl.kernel(
        _body,
        out_type=jax.ShapeDtypeStruct((BATCH, N), jnp.float32),
        mesh=mesh,
        compiler_params=pltpu.CompilerParams(needs_layout_passes=False),
        scratch_types=[
            pltpu.VMEM((N,), jnp.float32),
            pltpu.VMEM((N,), jnp.int32),
            pltpu.VMEM((N,), jnp.float32),
            pltpu.VMEM((HIST_WORDS,), jnp.int32),
            pltpu.VMEM((RING * CH,), jnp.float32),
            pltpu.SemaphoreType.DMA,
            pltpu.SemaphoreType.DMA,
        ],
    )
    return f(x, duty_cycles)
